# Initial kernel scaffold; baseline (speedup 1.0000x reference)
import jax
import jax.numpy as jnp
from jax.experimental import pallas as pl


def kernel(inputs, pos_table):
    raise NotImplementedError("write your pallas kernel here")


# speedup vs baseline: 1.8064x; 1.8064x over previous
import jax
import jax.numpy as jnp
from jax.experimental import pallas as pl
from jax.experimental.pallas import tpu as pltpu

BATCH = 4
SEQ = 8192
DIM = 768
BLOCK_S = 512


def _add_kernel(x_ref, p_ref, o_ref):
    o_ref[...] = x_ref[...] + p_ref[...]


def kernel(inputs, pos_table):
    grid = (SEQ // BLOCK_S,)
    return pl.pallas_call(
        _add_kernel,
        grid=grid,
        in_specs=[
            pl.BlockSpec((BATCH, BLOCK_S, DIM), lambda i: (0, i, 0)),
            pl.BlockSpec((BLOCK_S, DIM), lambda i: (i, 0)),
        ],
        out_specs=pl.BlockSpec((BATCH, BLOCK_S, DIM), lambda i: (0, i, 0)),
        out_shape=jax.ShapeDtypeStruct((BATCH, SEQ, DIM), jnp.float32),
        compiler_params=pltpu.CompilerParams(
            dimension_semantics=("arbitrary",),
        ),
    )(inputs, pos_table)


# speedup vs baseline: 1.8106x; 1.0024x over previous
import jax
import jax.numpy as jnp
from jax.experimental import pallas as pl
from jax.experimental.pallas import tpu as pltpu

BATCH = 4
SEQ = 8192
DIM = 768
BLOCK_S = 1024


def _add_kernel(x_ref, p_ref, o_ref):
    o_ref[...] = x_ref[...] + p_ref[...]


def kernel(inputs, pos_table):
    grid = (SEQ // BLOCK_S,)
    return pl.pallas_call(
        _add_kernel,
        grid=grid,
        in_specs=[
            pl.BlockSpec((BATCH, BLOCK_S, DIM), lambda i: (0, i, 0)),
            pl.BlockSpec((BLOCK_S, DIM), lambda i: (i, 0)),
        ],
        out_specs=pl.BlockSpec((BATCH, BLOCK_S, DIM), lambda i: (0, i, 0)),
        out_shape=jax.ShapeDtypeStruct((BATCH, SEQ, DIM), jnp.float32),
        compiler_params=pltpu.CompilerParams(
            dimension_semantics=("arbitrary",),
        ),
    )(inputs, pos_table)
